# baseline (speedup 1.0000x reference)
import functools
import jax
import jax.numpy as jnp
from jax import lax
from jax.experimental import pallas as pl
from jax.experimental.pallas import tpu as pltpu
from jax.experimental.pallas import tpu_sc as plsc

N = 10000
E = 320000
D = 128
N_PAD = 10240
NW = 32
K = 80
EPT = 10160
E_PAD = EPT * NW
C = EPT // K
RPT = N_PAD // 16

_mesh = plsc.VectorSubcoreMesh(core_axis_name="c", subcore_axis_name="s")


def _deg_body(dst_hbm, zeros_hbm, out_hbm, dst_v, deg_l, acc_v, tmp_v, stage_sh):
    cid = lax.axis_index("c")
    sid = lax.axis_index("s")
    wid = sid * 2 + cid
    pltpu.sync_copy(zeros_hbm, deg_l)
    pltpu.sync_copy(dst_hbm.at[wid], dst_v)
    ones = jnp.ones((16,), jnp.float32)

    def body(i, carry):
        for j in range(K // 16):
            dv = dst_v[i, pl.ds(j * 16, 16)]
            plsc.addupdate_scatter(deg_l, [dv], ones)
        return carry

    lax.fori_loop(0, C, body, 0)
    pltpu.sync_copy(deg_l, stage_sh.at[sid])
    plsc.subcore_barrier()
    r0 = sid * RPT
    pltpu.sync_copy(stage_sh.at[0, pl.ds(r0, RPT)], acc_v)
    for t2 in range(1, 16):
        pltpu.sync_copy(stage_sh.at[t2, pl.ds(r0, RPT)], tmp_v)

        def radd(i, carry):
            o = i * 16
            acc_v[pl.ds(o, 16)] = acc_v[pl.ds(o, 16)] + tmp_v[pl.ds(o, 16)]
            return carry

        lax.fori_loop(0, RPT // 16, radd, 0)
    pltpu.sync_copy(acc_v, out_hbm.at[cid, pl.ds(r0, RPT)])


_deg_call = functools.partial(
    pl.kernel,
    out_type=jax.ShapeDtypeStruct((2, N_PAD), jnp.float32),
    mesh=_mesh,
    compiler_params=pltpu.CompilerParams(needs_layout_passes=False),
    scratch_types=[
        pltpu.VMEM((C, K), jnp.int32),
        pltpu.VMEM((N_PAD,), jnp.float32),
        pltpu.VMEM((RPT,), jnp.float32),
        pltpu.VMEM((RPT,), jnp.float32),
        pltpu.VMEM_SHARED((16, N_PAD), jnp.float32),
    ],
)(_deg_body)


NBUF = 4
NIDX = 6


def _scat_body(g_hbm, src_hbm, dst_hbm, zeros_hbm, out_hbm,
               src_v, dst_v, rows_v, acc_sh, sem_g, sem_s, sem_i):
    cid = lax.axis_index("c")
    sid = lax.axis_index("s")
    wid = sid * 2 + cid
    pltpu.sync_copy(zeros_hbm, acc_sh.at[pl.ds(sid * RPT, RPT)])

    def idx_start(i, s):
        pltpu.async_copy(src_hbm.at[wid, i], src_v.at[s], sem_i.at[s])
        pltpu.async_copy(dst_hbm.at[wid, i], dst_v.at[s], sem_i.at[s])

    def idx_wait(i, s):
        pltpu.make_async_copy(src_hbm.at[wid, i], src_v.at[s],
                              sem_i.at[s]).wait()
        pltpu.make_async_copy(dst_hbm.at[wid, i], dst_v.at[s],
                              sem_i.at[s]).wait()

    def gather(i, b):
        pltpu.async_copy(g_hbm.at[src_v.at[lax.rem(i, NIDX)]], rows_v.at[b],
                         sem_g.at[b])

    def wait_g(i, b):
        pltpu.make_async_copy(g_hbm.at[src_v.at[lax.rem(i, NIDX)]],
                              rows_v.at[b], sem_g.at[b]).wait()

    def scat(i, b):
        pltpu.async_copy(rows_v.at[b], acc_sh.at[dst_v.at[lax.rem(i, NIDX)]],
                         sem_s.at[b], add=True)

    def wait_s(i, b):
        pltpu.make_async_copy(rows_v.at[b],
                              acc_sh.at[dst_v.at[lax.rem(i, NIDX)]],
                              sem_s.at[b]).wait()

    for j in range(5):
        idx_start(j, j)
    for j in range(3):
        idx_wait(j, j)
        gather(j, j)
    plsc.subcore_barrier()

    def body(i, carry):
        b = lax.rem(i, NBUF)
        wait_g(i, b)
        scat(i, b)

        @pl.when(i >= 1)
        def _():
            wait_s(i - 1, lax.rem(i - 1, NBUF))

        @pl.when(i + 5 < C)
        def _():
            idx_start(i + 5, lax.rem(i + 5, NIDX))

        @pl.when(i + 3 < C)
        def _():
            idx_wait(i + 3, lax.rem(i + 3, NIDX))
            gather(i + 3, lax.rem(i + 3, NBUF))

        return carry

    lax.fori_loop(0, C, body, 0)
    wait_s(C - 1, lax.rem(C - 1, NBUF))
    plsc.subcore_barrier()
    pltpu.sync_copy(acc_sh.at[pl.ds(sid * RPT, RPT)],
                    out_hbm.at[cid, pl.ds(sid * RPT, RPT)])


_scat_call = functools.partial(
    pl.kernel,
    out_type=jax.ShapeDtypeStruct((2, N_PAD, D), jnp.float32),
    mesh=_mesh,
    scratch_types=[
        pltpu.VMEM((NIDX, K), jnp.int32),
        pltpu.VMEM((NIDX, K), jnp.int32),
        pltpu.VMEM((NBUF, K, D), jnp.float32),
        pltpu.VMEM_SHARED((N_PAD, D), jnp.float32),
        pltpu.SemaphoreType.DMA((NBUF,)),
        pltpu.SemaphoreType.DMA((NBUF,)),
        pltpu.SemaphoreType.DMA((NIDX,)),
    ],
)(_scat_body)


RB = 512
_GRID = N_PAD // RB


def _mm1_body(deg_ref, x_ref, w_ref, g_ref, dinv_ref):
    deg = deg_ref[0] + deg_ref[1] + 1.0
    dinv = lax.rsqrt(deg)
    h = jnp.dot(x_ref[...], w_ref[...], preferred_element_type=jnp.float32)
    g_ref[...] = h * dinv
    dinv_ref[...] = dinv


def _mm1(deg_p, x_p, W1):
    return pl.pallas_call(
        _mm1_body,
        grid=(_GRID,),
        in_specs=[
            pl.BlockSpec((2, RB, 1), lambda i: (0, i, 0)),
            pl.BlockSpec((RB, D), lambda i: (i, 0)),
            pl.BlockSpec((D, D), lambda i: (0, 0)),
        ],
        out_specs=[
            pl.BlockSpec((RB, D), lambda i: (i, 0)),
            pl.BlockSpec((RB, 1), lambda i: (i, 0)),
        ],
        out_shape=[
            jax.ShapeDtypeStruct((N_PAD, D), jnp.float32),
            jax.ShapeDtypeStruct((N_PAD, 1), jnp.float32),
        ],
    )(deg_p, x_p, W1)


def _mid_body(p_ref, g_ref, dinv_ref, b_ref, a_ref, w_ref, g2_ref):
    dinv = dinv_ref[...]
    s = dinv * (p_ref[0] + p_ref[1] + g_ref[...]) + b_ref[...]
    aa = a_ref[0, 0]
    z = jnp.where(s >= 0, s, aa * s)
    h2 = jnp.dot(z, w_ref[...], preferred_element_type=jnp.float32)
    g2_ref[...] = h2 * dinv


def _mid(p1, g1, dinv, b1r, ar, W2):
    return pl.pallas_call(
        _mid_body,
        grid=(_GRID,),
        in_specs=[
            pl.BlockSpec((2, RB, D), lambda i: (0, i, 0)),
            pl.BlockSpec((RB, D), lambda i: (i, 0)),
            pl.BlockSpec((RB, 1), lambda i: (i, 0)),
            pl.BlockSpec((1, D), lambda i: (0, 0)),
            pl.BlockSpec((1, 1), lambda i: (0, 0)),
            pl.BlockSpec((D, D), lambda i: (0, 0)),
        ],
        out_specs=pl.BlockSpec((RB, D), lambda i: (i, 0)),
        out_shape=jax.ShapeDtypeStruct((N_PAD, D), jnp.float32),
    )(p1, g1, dinv, b1r, ar, W2)


def _fin_body(p_ref, g_ref, dinv_ref, b_ref, a_ref, out_ref):
    dinv = dinv_ref[...]
    s = dinv * (p_ref[0] + p_ref[1] + g_ref[...]) + b_ref[...]
    aa = a_ref[0, 0]
    out_ref[...] = jnp.where(s >= 0, s, aa * s)


def _fin(p2, g2, dinv, b2r, ar):
    return pl.pallas_call(
        _fin_body,
        grid=(_GRID,),
        in_specs=[
            pl.BlockSpec((2, RB, D), lambda i: (0, i, 0)),
            pl.BlockSpec((RB, D), lambda i: (i, 0)),
            pl.BlockSpec((RB, 1), lambda i: (i, 0)),
            pl.BlockSpec((1, D), lambda i: (0, 0)),
            pl.BlockSpec((1, 1), lambda i: (0, 0)),
        ],
        out_specs=pl.BlockSpec((RB, D), lambda i: (i, 0)),
        out_shape=jax.ShapeDtypeStruct((N_PAD, D), jnp.float32),
    )(p2, g2, dinv, b2r, ar)


def kernel(x, edge_index, W1, b1, W2, b2, a):
    src = edge_index[0]
    dst = edge_index[1]
    pad_idx = jnp.full((E_PAD - E,), N, dtype=jnp.int32)
    src_p = jnp.concatenate([src, pad_idx]).reshape(NW, C, K)
    dst_p = jnp.concatenate([dst, pad_idx]).reshape(NW, C, K)
    x_p = jnp.concatenate([x, jnp.zeros((N_PAD - N, D), x.dtype)])
    zerosN = jnp.zeros((N_PAD,), jnp.float32)
    zerosD = jnp.zeros((RPT, D), jnp.float32)
    b1r = b1.reshape(1, D)
    b2r = b2.reshape(1, D)
    ar = a.reshape(1, 1)

    deg_p = _deg_call(dst_p, zerosN).reshape(2, N_PAD, 1)
    g1, dinv = _mm1(deg_p, x_p, W1)
    p1 = _scat_call(g1, src_p, dst_p, zerosD)
    g2 = _mid(p1, g1, dinv, b1r, ar, W2)
    p2 = _scat_call(g2, src_p, dst_p, zerosD)
    out = _fin(p2, g2, dinv, b2r, ar)
    return out[:N]


# speedup vs baseline: 1.2901x; 1.2901x over previous
import functools
import jax
import jax.numpy as jnp
from jax import lax
from jax.experimental import pallas as pl
from jax.experimental.pallas import tpu as pltpu
from jax.experimental.pallas import tpu_sc as plsc

N = 10000
E = 320000
D = 128
N_PAD = 10240
NW = 32
K = 128
EPT = 10112
E_PAD = EPT * NW
C = EPT // K
RPT = N_PAD // 16
N_ACC = 10048
HALF = N_ACC // 8

_mesh = plsc.VectorSubcoreMesh(core_axis_name="c", subcore_axis_name="s")


def _deg_body(dst_hbm, zeros_hbm, out_hbm, dst_v, deg_l, acc_v, tmp_v, stage_sh):
    cid = lax.axis_index("c")
    sid = lax.axis_index("s")
    wid = sid * 2 + cid
    pltpu.sync_copy(zeros_hbm, deg_l)
    pltpu.sync_copy(dst_hbm.at[wid], dst_v)
    ones = jnp.ones((16,), jnp.float32)

    def body(i, carry):
        for j in range(K // 16):
            dv = dst_v[i, pl.ds(j * 16, 16)]
            plsc.addupdate_scatter(deg_l, [dv], ones)
        return carry

    lax.fori_loop(0, C, body, 0)
    pltpu.sync_copy(deg_l, stage_sh.at[sid])
    plsc.subcore_barrier()
    r0 = sid * RPT
    pltpu.sync_copy(stage_sh.at[0, pl.ds(r0, RPT)], acc_v)
    for t2 in range(1, 16):
        pltpu.sync_copy(stage_sh.at[t2, pl.ds(r0, RPT)], tmp_v)

        def radd(i, carry):
            o = i * 16
            acc_v[pl.ds(o, 16)] = acc_v[pl.ds(o, 16)] + tmp_v[pl.ds(o, 16)]
            return carry

        lax.fori_loop(0, RPT // 16, radd, 0)
    pltpu.sync_copy(acc_v, out_hbm.at[cid, pl.ds(r0, RPT)])


_deg_call = functools.partial(
    pl.kernel,
    out_type=jax.ShapeDtypeStruct((2, N_PAD), jnp.float32),
    mesh=_mesh,
    compiler_params=pltpu.CompilerParams(needs_layout_passes=False),
    scratch_types=[
        pltpu.VMEM((C, K), jnp.int32),
        pltpu.VMEM((N_PAD,), jnp.float32),
        pltpu.VMEM((RPT,), jnp.float32),
        pltpu.VMEM((RPT,), jnp.float32),
        pltpu.VMEM_SHARED((16, N_PAD), jnp.float32),
    ],
)(_deg_body)


NBUF = 3
NIDX = 5


def _scat_body(g_hbm, idx_hbm, zeros_hbm, out_hbm,
               idx_v, rows_v, acc_sh, sem_g, sem_s, sem_i):
    cid = lax.axis_index("c")
    sid = lax.axis_index("s")
    wid = sid * 2 + cid

    @pl.when(sid < 8)
    def _():
        pltpu.sync_copy(zeros_hbm, acc_sh.at[pl.ds(sid * HALF, HALF)])

    def idx_start(i, s):
        pltpu.async_copy(idx_hbm.at[wid, i], idx_v.at[s], sem_i.at[s])

    def idx_wait(i, s):
        pltpu.make_async_copy(idx_hbm.at[wid, i], idx_v.at[s],
                              sem_i.at[s]).wait()

    def gather(i, b):
        pltpu.async_copy(g_hbm.at[idx_v.at[lax.rem(i, NIDX), 0]],
                         rows_v.at[b], sem_g.at[b])

    def wait_g(i, b):
        pltpu.make_async_copy(g_hbm.at[idx_v.at[lax.rem(i, NIDX), 0]],
                              rows_v.at[b], sem_g.at[b]).wait()

    def scat(i, b):
        pltpu.async_copy(rows_v.at[b],
                         acc_sh.at[idx_v.at[lax.rem(i, NIDX), 1]],
                         sem_s.at[b], add=True)

    def wait_s(i, b):
        pltpu.make_async_copy(rows_v.at[b],
                              acc_sh.at[idx_v.at[lax.rem(i, NIDX), 1]],
                              sem_s.at[b]).wait()

    for j in range(4):
        idx_start(j, j)
    for j in range(2):
        idx_wait(j, j)
        gather(j, j)
    plsc.subcore_barrier()

    def body(i, carry):
        b = lax.rem(i, NBUF)
        wait_g(i, b)
        scat(i, b)

        @pl.when(i >= 1)
        def _():
            wait_s(i - 1, lax.rem(i - 1, NBUF))

        @pl.when(i + 4 < C)
        def _():
            idx_start(i + 4, lax.rem(i + 4, NIDX))

        @pl.when(i + 2 < C)
        def _():
            idx_wait(i + 2, lax.rem(i + 2, NIDX))
            gather(i + 2, lax.rem(i + 2, NBUF))

        return carry

    lax.fori_loop(0, C, body, 0)
    wait_s(C - 1, lax.rem(C - 1, NBUF))
    plsc.subcore_barrier()

    @pl.when(sid < 8)
    def _():
        pltpu.sync_copy(acc_sh.at[pl.ds(sid * HALF, HALF)],
                        out_hbm.at[cid, pl.ds(sid * HALF, HALF)])


_scat_call = functools.partial(
    pl.kernel,
    out_type=jax.ShapeDtypeStruct((2, N_PAD, D), jnp.float32),
    mesh=_mesh,
    scratch_types=[
        pltpu.VMEM((NIDX, 2, K), jnp.int32),
        pltpu.VMEM((NBUF, K, D), jnp.float32),
        pltpu.VMEM_SHARED((N_ACC, D), jnp.float32),
        pltpu.SemaphoreType.DMA((NBUF,)),
        pltpu.SemaphoreType.DMA((NBUF,)),
        pltpu.SemaphoreType.DMA((NIDX,)),
    ],
)(_scat_body)


RB = 512
_GRID = N_PAD // RB


def _mm1_body(deg_ref, x_ref, w_ref, g_ref, dinv_ref):
    deg = deg_ref[0] + deg_ref[1] + 1.0
    dinv = lax.rsqrt(deg)
    h = jnp.dot(x_ref[...], w_ref[...], preferred_element_type=jnp.float32)
    g_ref[...] = h * dinv
    dinv_ref[...] = dinv


def _mm1(deg_p, x_p, W1):
    return pl.pallas_call(
        _mm1_body,
        grid=(_GRID,),
        in_specs=[
            pl.BlockSpec((2, RB, 1), lambda i: (0, i, 0)),
            pl.BlockSpec((RB, D), lambda i: (i, 0)),
            pl.BlockSpec((D, D), lambda i: (0, 0)),
        ],
        out_specs=[
            pl.BlockSpec((RB, D), lambda i: (i, 0)),
            pl.BlockSpec((RB, 1), lambda i: (i, 0)),
        ],
        out_shape=[
            jax.ShapeDtypeStruct((N_PAD, D), jnp.float32),
            jax.ShapeDtypeStruct((N_PAD, 1), jnp.float32),
        ],
    )(deg_p, x_p, W1)


def _mid_body(p_ref, g_ref, dinv_ref, b_ref, a_ref, w_ref, g2_ref):
    dinv = dinv_ref[...]
    s = dinv * (p_ref[0] + p_ref[1] + g_ref[...]) + b_ref[...]
    aa = a_ref[0, 0]
    z = jnp.where(s >= 0, s, aa * s)
    h2 = jnp.dot(z, w_ref[...], preferred_element_type=jnp.float32)
    g2_ref[...] = h2 * dinv


def _mid(p1, g1, dinv, b1r, ar, W2):
    return pl.pallas_call(
        _mid_body,
        grid=(_GRID,),
        in_specs=[
            pl.BlockSpec((2, RB, D), lambda i: (0, i, 0)),
            pl.BlockSpec((RB, D), lambda i: (i, 0)),
            pl.BlockSpec((RB, 1), lambda i: (i, 0)),
            pl.BlockSpec((1, D), lambda i: (0, 0)),
            pl.BlockSpec((1, 1), lambda i: (0, 0)),
            pl.BlockSpec((D, D), lambda i: (0, 0)),
        ],
        out_specs=pl.BlockSpec((RB, D), lambda i: (i, 0)),
        out_shape=jax.ShapeDtypeStruct((N_PAD, D), jnp.float32),
    )(p1, g1, dinv, b1r, ar, W2)


def _fin_body(p_ref, g_ref, dinv_ref, b_ref, a_ref, out_ref):
    dinv = dinv_ref[...]
    s = dinv * (p_ref[0] + p_ref[1] + g_ref[...]) + b_ref[...]
    aa = a_ref[0, 0]
    out_ref[...] = jnp.where(s >= 0, s, aa * s)


def _fin(p2, g2, dinv, b2r, ar):
    return pl.pallas_call(
        _fin_body,
        grid=(_GRID,),
        in_specs=[
            pl.BlockSpec((2, RB, D), lambda i: (0, i, 0)),
            pl.BlockSpec((RB, D), lambda i: (i, 0)),
            pl.BlockSpec((RB, 1), lambda i: (i, 0)),
            pl.BlockSpec((1, D), lambda i: (0, 0)),
            pl.BlockSpec((1, 1), lambda i: (0, 0)),
        ],
        out_specs=pl.BlockSpec((RB, D), lambda i: (i, 0)),
        out_shape=jax.ShapeDtypeStruct((N_PAD, D), jnp.float32),
    )(p2, g2, dinv, b2r, ar)


def kernel(x, edge_index, W1, b1, W2, b2, a):
    src = edge_index[0]
    dst = edge_index[1]
    pad_idx = jnp.full((E_PAD - E,), N, dtype=jnp.int32)
    src_p = jnp.concatenate([src, pad_idx]).reshape(NW, C, K)
    dst_p = jnp.concatenate([dst, pad_idx]).reshape(NW, C, K)
    idx_p = jnp.stack([src_p, dst_p], axis=2)
    x_p = jnp.concatenate([x, jnp.zeros((N_PAD - N, D), x.dtype)])
    zerosN = jnp.zeros((N_PAD,), jnp.float32)
    zerosD = jnp.zeros((HALF, D), jnp.float32)
    b1r = b1.reshape(1, D)
    b2r = b2.reshape(1, D)
    ar = a.reshape(1, 1)

    deg_p = _deg_call(dst_p, zerosN).reshape(2, N_PAD, 1)
    g1, dinv = _mm1(deg_p, x_p, W1)
    p1 = _scat_call(g1, idx_p, zerosD)
    g2 = _mid(p1, g1, dinv, b1r, ar, W2)
    p2 = _scat_call(g2, idx_p, zerosD)
    out = _fin(p2, g2, dinv, b2r, ar)
    return out[:N]


# speedup vs baseline: 1.5191x; 1.1775x over previous
import functools
import jax
import jax.numpy as jnp
from jax import lax
from jax.experimental import pallas as pl
from jax.experimental.pallas import tpu as pltpu
from jax.experimental.pallas import tpu_sc as plsc

N = 10000
E = 320000
D = 128
N_PAD = 10240
NW = 32
K = 112
EPT = 10080
E_PAD = EPT * NW
C = EPT // K
RPT = N_PAD // 16
N_ACC = 10048
HALF = N_ACC // 8

_mesh = plsc.VectorSubcoreMesh(core_axis_name="c", subcore_axis_name="s")


def _deg_body(dst_hbm, zeros_hbm, out_hbm, dst_v, deg_l, acc_v, tmp_v, stage_sh):
    cid = lax.axis_index("c")
    sid = lax.axis_index("s")
    wid = sid * 2 + cid
    pltpu.sync_copy(zeros_hbm, deg_l)
    pltpu.sync_copy(dst_hbm.at[wid], dst_v)
    ones = jnp.ones((16,), jnp.float32)

    def body(i, carry):
        for j in range(K // 16):
            dv = dst_v[i, pl.ds(j * 16, 16)]
            plsc.addupdate_scatter(deg_l, [dv], ones)
        return carry

    lax.fori_loop(0, C, body, 0)
    pltpu.sync_copy(deg_l, stage_sh.at[sid])
    plsc.subcore_barrier()
    r0 = sid * RPT
    pltpu.sync_copy(stage_sh.at[0, pl.ds(r0, RPT)], acc_v)
    for t2 in range(1, 16):
        pltpu.sync_copy(stage_sh.at[t2, pl.ds(r0, RPT)], tmp_v)

        def radd(i, carry):
            o = i * 16
            acc_v[pl.ds(o, 16)] = acc_v[pl.ds(o, 16)] + tmp_v[pl.ds(o, 16)]
            return carry

        lax.fori_loop(0, RPT // 16, radd, 0)
    pltpu.sync_copy(acc_v, out_hbm.at[cid, pl.ds(r0, RPT)])


_deg_call = functools.partial(
    pl.kernel,
    out_type=jax.ShapeDtypeStruct((2, N_PAD), jnp.float32),
    mesh=_mesh,
    compiler_params=pltpu.CompilerParams(needs_layout_passes=False),
    scratch_types=[
        pltpu.VMEM((C, K), jnp.int32),
        pltpu.VMEM((N_PAD,), jnp.float32),
        pltpu.VMEM((RPT,), jnp.float32),
        pltpu.VMEM((RPT,), jnp.float32),
        pltpu.VMEM_SHARED((16, N_PAD), jnp.float32),
    ],
)(_deg_body)


NBUF = 3
NIDX = 5


def _scat_body(g_hbm, idx_hbm, zeros_hbm, out_hbm,
               idx_v, rows_v, acc_sh, sem_g, sem_s, sem_i):
    cid = lax.axis_index("c")
    sid = lax.axis_index("s")
    wid = sid * 2 + cid

    @pl.when(sid < 8)
    def _():
        pltpu.sync_copy(zeros_hbm, acc_sh.at[pl.ds(sid * HALF, HALF)])

    def idx_start(i, s):
        pltpu.async_copy(idx_hbm.at[wid, i], idx_v.at[s], sem_i.at[s])

    def idx_wait(i, s):
        pltpu.make_async_copy(idx_hbm.at[wid, i], idx_v.at[s],
                              sem_i.at[s]).wait()

    def gather(i, b):
        pltpu.async_copy(g_hbm.at[idx_v.at[lax.rem(i, NIDX), 0]],
                         rows_v.at[b], sem_g.at[b])

    def wait_g(i, b):
        pltpu.make_async_copy(g_hbm.at[idx_v.at[lax.rem(i, NIDX), 0]],
                              rows_v.at[b], sem_g.at[b]).wait()

    def scat(i, b):
        pltpu.async_copy(rows_v.at[b],
                         acc_sh.at[idx_v.at[lax.rem(i, NIDX), 1]],
                         sem_s.at[b], add=True)

    def wait_s(i, b):
        pltpu.make_async_copy(rows_v.at[b],
                              acc_sh.at[idx_v.at[lax.rem(i, NIDX), 1]],
                              sem_s.at[b]).wait()

    for j in range(4):
        idx_start(j, j)
    for j in range(2):
        idx_wait(j, j)
        gather(j, j)
    plsc.subcore_barrier()

    def body(i, carry):
        b = lax.rem(i, NBUF)
        wait_g(i, b)
        scat(i, b)

        @pl.when(i >= 1)
        def _():
            wait_s(i - 1, lax.rem(i - 1, NBUF))

        @pl.when(i + 4 < C)
        def _():
            idx_start(i + 4, lax.rem(i + 4, NIDX))

        @pl.when(i + 2 < C)
        def _():
            idx_wait(i + 2, lax.rem(i + 2, NIDX))
            gather(i + 2, lax.rem(i + 2, NBUF))

        return carry

    lax.fori_loop(0, C, body, 0)
    wait_s(C - 1, lax.rem(C - 1, NBUF))
    plsc.subcore_barrier()

    @pl.when(sid < 8)
    def _():
        pltpu.sync_copy(acc_sh.at[pl.ds(sid * HALF, HALF)],
                        out_hbm.at[cid, pl.ds(sid * HALF, HALF)])


_scat_call = functools.partial(
    pl.kernel,
    out_type=jax.ShapeDtypeStruct((2, N_PAD, D), jnp.float32),
    mesh=_mesh,
    scratch_types=[
        pltpu.VMEM((NIDX, 2, K), jnp.int32),
        pltpu.VMEM((NBUF, K, D), jnp.float32),
        pltpu.VMEM_SHARED((N_ACC, D), jnp.float32),
        pltpu.SemaphoreType.DMA((NBUF,)),
        pltpu.SemaphoreType.DMA((NBUF,)),
        pltpu.SemaphoreType.DMA((NIDX,)),
    ],
)(_scat_body)


RB = 512
_GRID = N_PAD // RB


def _mm1_body(deg_ref, x_ref, w_ref, g_ref, dinv_ref):
    deg = deg_ref[0] + deg_ref[1] + 1.0
    dinv = lax.rsqrt(deg)
    h = jnp.dot(x_ref[...], w_ref[...], preferred_element_type=jnp.float32)
    g_ref[...] = h * dinv
    dinv_ref[...] = dinv


def _mm1(deg_p, x_p, W1):
    return pl.pallas_call(
        _mm1_body,
        grid=(_GRID,),
        in_specs=[
            pl.BlockSpec((2, RB, 1), lambda i: (0, i, 0)),
            pl.BlockSpec((RB, D), lambda i: (i, 0)),
            pl.BlockSpec((D, D), lambda i: (0, 0)),
        ],
        out_specs=[
            pl.BlockSpec((RB, D), lambda i: (i, 0)),
            pl.BlockSpec((RB, 1), lambda i: (i, 0)),
        ],
        out_shape=[
            jax.ShapeDtypeStruct((N_PAD, D), jnp.float32),
            jax.ShapeDtypeStruct((N_PAD, 1), jnp.float32),
        ],
    )(deg_p, x_p, W1)


def _mid_body(p_ref, g_ref, dinv_ref, b_ref, a_ref, w_ref, g2_ref):
    dinv = dinv_ref[...]
    s = dinv * (p_ref[0] + p_ref[1] + g_ref[...]) + b_ref[...]
    aa = a_ref[0, 0]
    z = jnp.where(s >= 0, s, aa * s)
    h2 = jnp.dot(z, w_ref[...], preferred_element_type=jnp.float32)
    g2_ref[...] = h2 * dinv


def _mid(p1, g1, dinv, b1r, ar, W2):
    return pl.pallas_call(
        _mid_body,
        grid=(_GRID,),
        in_specs=[
            pl.BlockSpec((2, RB, D), lambda i: (0, i, 0)),
            pl.BlockSpec((RB, D), lambda i: (i, 0)),
            pl.BlockSpec((RB, 1), lambda i: (i, 0)),
            pl.BlockSpec((1, D), lambda i: (0, 0)),
            pl.BlockSpec((1, 1), lambda i: (0, 0)),
            pl.BlockSpec((D, D), lambda i: (0, 0)),
        ],
        out_specs=pl.BlockSpec((RB, D), lambda i: (i, 0)),
        out_shape=jax.ShapeDtypeStruct((N_PAD, D), jnp.float32),
    )(p1, g1, dinv, b1r, ar, W2)


def _fin_body(p_ref, g_ref, dinv_ref, b_ref, a_ref, out_ref):
    dinv = dinv_ref[...]
    s = dinv * (p_ref[0] + p_ref[1] + g_ref[...]) + b_ref[...]
    aa = a_ref[0, 0]
    out_ref[...] = jnp.where(s >= 0, s, aa * s)


def _fin(p2, g2, dinv, b2r, ar):
    return pl.pallas_call(
        _fin_body,
        grid=(_GRID,),
        in_specs=[
            pl.BlockSpec((2, RB, D), lambda i: (0, i, 0)),
            pl.BlockSpec((RB, D), lambda i: (i, 0)),
            pl.BlockSpec((RB, 1), lambda i: (i, 0)),
            pl.BlockSpec((1, D), lambda i: (0, 0)),
            pl.BlockSpec((1, 1), lambda i: (0, 0)),
        ],
        out_specs=pl.BlockSpec((RB, D), lambda i: (i, 0)),
        out_shape=jax.ShapeDtypeStruct((N_PAD, D), jnp.float32),
    )(p2, g2, dinv, b2r, ar)


def kernel(x, edge_index, W1, b1, W2, b2, a):
    src = edge_index[0]
    dst = edge_index[1]
    pad_idx = jnp.full((E_PAD - E,), N, dtype=jnp.int32)
    src_p = jnp.concatenate([src, pad_idx]).reshape(NW, C, K)
    dst_p = jnp.concatenate([dst, pad_idx]).reshape(NW, C, K)
    idx_p = jnp.stack([src_p, dst_p], axis=2)
    x_p = jnp.concatenate([x, jnp.zeros((N_PAD - N, D), x.dtype)])
    zerosN = jnp.zeros((N_PAD,), jnp.float32)
    zerosD = jnp.zeros((HALF, D), jnp.float32)
    b1r = b1.reshape(1, D)
    b2r = b2.reshape(1, D)
    ar = a.reshape(1, 1)

    deg_p = _deg_call(dst_p, zerosN).reshape(2, N_PAD, 1)
    g1, dinv = _mm1(deg_p, x_p, W1)
    p1 = _scat_call(g1, idx_p, zerosD)
    g2 = _mid(p1, g1, dinv, b1r, ar, W2)
    p2 = _scat_call(g2, idx_p, zerosD)
    out = _fin(p2, g2, dinv, b2r, ar)
    return out[:N]


# speedup vs baseline: 1.5769x; 1.0380x over previous
import functools
import jax
import jax.numpy as jnp
from jax import lax
from jax.experimental import pallas as pl
from jax.experimental.pallas import tpu as pltpu
from jax.experimental.pallas import tpu_sc as plsc

N = 10000
E = 320000
D = 128
N_PAD = 10240
NW = 32
K = 96
EPT = 10080
E_PAD = EPT * NW
C = EPT // K
RPT = N_PAD // 16
N_ACC = 10048
HALF = N_ACC // 8

_mesh = plsc.VectorSubcoreMesh(core_axis_name="c", subcore_axis_name="s")


def _deg_body(dst_hbm, zeros_hbm, out_hbm, dst_v, deg_l, acc_v, tmp_v, stage_sh):
    cid = lax.axis_index("c")
    sid = lax.axis_index("s")
    wid = sid * 2 + cid
    pltpu.sync_copy(zeros_hbm, deg_l)
    pltpu.sync_copy(dst_hbm.at[wid], dst_v)
    ones = jnp.ones((16,), jnp.float32)

    def body(i, carry):
        for j in range(K // 16):
            dv = dst_v[i, pl.ds(j * 16, 16)]
            plsc.addupdate_scatter(deg_l, [dv], ones)
        return carry

    lax.fori_loop(0, C, body, 0)
    pltpu.sync_copy(deg_l, stage_sh.at[sid])
    plsc.subcore_barrier()
    r0 = sid * RPT
    pltpu.sync_copy(stage_sh.at[0, pl.ds(r0, RPT)], acc_v)
    for t2 in range(1, 16):
        pltpu.sync_copy(stage_sh.at[t2, pl.ds(r0, RPT)], tmp_v)

        def radd(i, carry):
            o = i * 16
            acc_v[pl.ds(o, 16)] = acc_v[pl.ds(o, 16)] + tmp_v[pl.ds(o, 16)]
            return carry

        lax.fori_loop(0, RPT // 16, radd, 0)
    pltpu.sync_copy(acc_v, out_hbm.at[cid, pl.ds(r0, RPT)])


_deg_call = functools.partial(
    pl.kernel,
    out_type=jax.ShapeDtypeStruct((2, N_PAD), jnp.float32),
    mesh=_mesh,
    compiler_params=pltpu.CompilerParams(needs_layout_passes=False),
    scratch_types=[
        pltpu.VMEM((C, K), jnp.int32),
        pltpu.VMEM((N_PAD,), jnp.float32),
        pltpu.VMEM((RPT,), jnp.float32),
        pltpu.VMEM((RPT,), jnp.float32),
        pltpu.VMEM_SHARED((16, N_PAD), jnp.float32),
    ],
)(_deg_body)


NBUF = 3
NIDX = 5


def _scat_body(g_hbm, idx_hbm, zeros_hbm, out_hbm,
               idx_v, rows_v, acc_sh, sem_g, sem_s, sem_i):
    cid = lax.axis_index("c")
    sid = lax.axis_index("s")
    wid = sid * 2 + cid

    @pl.when(sid < 8)
    def _():
        pltpu.sync_copy(zeros_hbm, acc_sh.at[pl.ds(sid * HALF, HALF)])

    def idx_start(i, s):
        pltpu.async_copy(idx_hbm.at[wid, i], idx_v.at[s], sem_i.at[s])

    def idx_wait(i, s):
        pltpu.make_async_copy(idx_hbm.at[wid, i], idx_v.at[s],
                              sem_i.at[s]).wait()

    def gather(i, b):
        pltpu.async_copy(g_hbm.at[idx_v.at[lax.rem(i, NIDX), 0]],
                         rows_v.at[b], sem_g.at[b])

    def wait_g(i, b):
        pltpu.make_async_copy(g_hbm.at[idx_v.at[lax.rem(i, NIDX), 0]],
                              rows_v.at[b], sem_g.at[b]).wait()

    def scat(i, b):
        pltpu.async_copy(rows_v.at[b],
                         acc_sh.at[idx_v.at[lax.rem(i, NIDX), 1]],
                         sem_s.at[b], add=True)

    def wait_s(i, b):
        pltpu.make_async_copy(rows_v.at[b],
                              acc_sh.at[idx_v.at[lax.rem(i, NIDX), 1]],
                              sem_s.at[b]).wait()

    for j in range(4):
        idx_start(j, j)
    for j in range(2):
        idx_wait(j, j)
        gather(j, j)
    plsc.subcore_barrier()

    def body(i, carry):
        b = lax.rem(i, NBUF)
        wait_g(i, b)
        scat(i, b)

        @pl.when(i >= 1)
        def _():
            wait_s(i - 1, lax.rem(i - 1, NBUF))

        @pl.when(i + 4 < C)
        def _():
            idx_start(i + 4, lax.rem(i + 4, NIDX))

        @pl.when(i + 2 < C)
        def _():
            idx_wait(i + 2, lax.rem(i + 2, NIDX))
            gather(i + 2, lax.rem(i + 2, NBUF))

        return carry

    lax.fori_loop(0, C, body, 0)
    wait_s(C - 1, lax.rem(C - 1, NBUF))
    plsc.subcore_barrier()

    @pl.when(sid < 8)
    def _():
        pltpu.sync_copy(acc_sh.at[pl.ds(sid * HALF, HALF)],
                        out_hbm.at[cid, pl.ds(sid * HALF, HALF)])


_scat_call = functools.partial(
    pl.kernel,
    out_type=jax.ShapeDtypeStruct((2, N_PAD, D), jnp.float32),
    mesh=_mesh,
    scratch_types=[
        pltpu.VMEM((NIDX, 2, K), jnp.int32),
        pltpu.VMEM((NBUF, K, D), jnp.float32),
        pltpu.VMEM_SHARED((N_ACC, D), jnp.float32),
        pltpu.SemaphoreType.DMA((NBUF,)),
        pltpu.SemaphoreType.DMA((NBUF,)),
        pltpu.SemaphoreType.DMA((NIDX,)),
    ],
)(_scat_body)


RB = 512
_GRID = N_PAD // RB


def _mm1_body(deg_ref, x_ref, w_ref, g_ref, dinv_ref):
    deg = deg_ref[0] + deg_ref[1] + 1.0
    dinv = lax.rsqrt(deg)
    h = jnp.dot(x_ref[...], w_ref[...], preferred_element_type=jnp.float32)
    g_ref[...] = h * dinv
    dinv_ref[...] = dinv


def _mm1(deg_p, x_p, W1):
    return pl.pallas_call(
        _mm1_body,
        grid=(_GRID,),
        in_specs=[
            pl.BlockSpec((2, RB, 1), lambda i: (0, i, 0)),
            pl.BlockSpec((RB, D), lambda i: (i, 0)),
            pl.BlockSpec((D, D), lambda i: (0, 0)),
        ],
        out_specs=[
            pl.BlockSpec((RB, D), lambda i: (i, 0)),
            pl.BlockSpec((RB, 1), lambda i: (i, 0)),
        ],
        out_shape=[
            jax.ShapeDtypeStruct((N_PAD, D), jnp.float32),
            jax.ShapeDtypeStruct((N_PAD, 1), jnp.float32),
        ],
    )(deg_p, x_p, W1)


def _mid_body(p_ref, g_ref, dinv_ref, b_ref, a_ref, w_ref, g2_ref):
    dinv = dinv_ref[...]
    s = dinv * (p_ref[0] + p_ref[1] + g_ref[...]) + b_ref[...]
    aa = a_ref[0, 0]
    z = jnp.where(s >= 0, s, aa * s)
    h2 = jnp.dot(z, w_ref[...], preferred_element_type=jnp.float32)
    g2_ref[...] = h2 * dinv


def _mid(p1, g1, dinv, b1r, ar, W2):
    return pl.pallas_call(
        _mid_body,
        grid=(_GRID,),
        in_specs=[
            pl.BlockSpec((2, RB, D), lambda i: (0, i, 0)),
            pl.BlockSpec((RB, D), lambda i: (i, 0)),
            pl.BlockSpec((RB, 1), lambda i: (i, 0)),
            pl.BlockSpec((1, D), lambda i: (0, 0)),
            pl.BlockSpec((1, 1), lambda i: (0, 0)),
            pl.BlockSpec((D, D), lambda i: (0, 0)),
        ],
        out_specs=pl.BlockSpec((RB, D), lambda i: (i, 0)),
        out_shape=jax.ShapeDtypeStruct((N_PAD, D), jnp.float32),
    )(p1, g1, dinv, b1r, ar, W2)


def _fin_body(p_ref, g_ref, dinv_ref, b_ref, a_ref, out_ref):
    dinv = dinv_ref[...]
    s = dinv * (p_ref[0] + p_ref[1] + g_ref[...]) + b_ref[...]
    aa = a_ref[0, 0]
    out_ref[...] = jnp.where(s >= 0, s, aa * s)


def _fin(p2, g2, dinv, b2r, ar):
    return pl.pallas_call(
        _fin_body,
        grid=(_GRID,),
        in_specs=[
            pl.BlockSpec((2, RB, D), lambda i: (0, i, 0)),
            pl.BlockSpec((RB, D), lambda i: (i, 0)),
            pl.BlockSpec((RB, 1), lambda i: (i, 0)),
            pl.BlockSpec((1, D), lambda i: (0, 0)),
            pl.BlockSpec((1, 1), lambda i: (0, 0)),
        ],
        out_specs=pl.BlockSpec((RB, D), lambda i: (i, 0)),
        out_shape=jax.ShapeDtypeStruct((N_PAD, D), jnp.float32),
    )(p2, g2, dinv, b2r, ar)


def kernel(x, edge_index, W1, b1, W2, b2, a):
    src = edge_index[0]
    dst = edge_index[1]
    pad_idx = jnp.full((E_PAD - E,), N, dtype=jnp.int32)
    src_p = jnp.concatenate([src, pad_idx]).reshape(NW, C, K)
    dst_p = jnp.concatenate([dst, pad_idx]).reshape(NW, C, K)
    idx_p = jnp.stack([src_p, dst_p], axis=2)
    x_p = jnp.concatenate([x, jnp.zeros((N_PAD - N, D), x.dtype)])
    zerosN = jnp.zeros((N_PAD,), jnp.float32)
    zerosD = jnp.zeros((HALF, D), jnp.float32)
    b1r = b1.reshape(1, D)
    b2r = b2.reshape(1, D)
    ar = a.reshape(1, 1)

    deg_p = _deg_call(dst_p, zerosN).reshape(2, N_PAD, 1)
    g1, dinv = _mm1(deg_p, x_p, W1)
    p1 = _scat_call(g1, idx_p, zerosD)
    g2 = _mid(p1, g1, dinv, b1r, ar, W2)
    p2 = _scat_call(g2, idx_p, zerosD)
    out = _fin(p2, g2, dinv, b2r, ar)
    return out[:N]


# speedup vs baseline: 1.6102x; 1.0211x over previous
import functools
import jax
import jax.numpy as jnp
from jax import lax
from jax.experimental import pallas as pl
from jax.experimental.pallas import tpu as pltpu
from jax.experimental.pallas import tpu_sc as plsc

N = 10000
E = 320000
D = 128
N_PAD = 10240
NW = 32
K = 80
EPT = 10080
E_PAD = EPT * NW
C = EPT // K
RPT = N_PAD // 16
N_ACC = 10048
HALF = N_ACC // 8

_mesh = plsc.VectorSubcoreMesh(core_axis_name="c", subcore_axis_name="s")


def _deg_body(dst_hbm, zeros_hbm, out_hbm, dst_v, deg_l, acc_v, tmp_v, stage_sh):
    cid = lax.axis_index("c")
    sid = lax.axis_index("s")
    wid = sid * 2 + cid
    pltpu.sync_copy(zeros_hbm, deg_l)
    pltpu.sync_copy(dst_hbm.at[wid], dst_v)
    ones = jnp.ones((16,), jnp.float32)

    def body(i, carry):
        for j in range(K // 16):
            dv = dst_v[i, pl.ds(j * 16, 16)]
            plsc.addupdate_scatter(deg_l, [dv], ones)
        return carry

    lax.fori_loop(0, C, body, 0)
    pltpu.sync_copy(deg_l, stage_sh.at[sid])
    plsc.subcore_barrier()
    r0 = sid * RPT
    pltpu.sync_copy(stage_sh.at[0, pl.ds(r0, RPT)], acc_v)
    for t2 in range(1, 16):
        pltpu.sync_copy(stage_sh.at[t2, pl.ds(r0, RPT)], tmp_v)

        def radd(i, carry):
            o = i * 16
            acc_v[pl.ds(o, 16)] = acc_v[pl.ds(o, 16)] + tmp_v[pl.ds(o, 16)]
            return carry

        lax.fori_loop(0, RPT // 16, radd, 0)
    pltpu.sync_copy(acc_v, out_hbm.at[cid, pl.ds(r0, RPT)])


_deg_call = functools.partial(
    pl.kernel,
    out_type=jax.ShapeDtypeStruct((2, N_PAD), jnp.float32),
    mesh=_mesh,
    compiler_params=pltpu.CompilerParams(needs_layout_passes=False),
    scratch_types=[
        pltpu.VMEM((C, K), jnp.int32),
        pltpu.VMEM((N_PAD,), jnp.float32),
        pltpu.VMEM((RPT,), jnp.float32),
        pltpu.VMEM((RPT,), jnp.float32),
        pltpu.VMEM_SHARED((16, N_PAD), jnp.float32),
    ],
)(_deg_body)


NBUF = 3
NIDX = 5


def _scat_body(g_hbm, idx_hbm, zeros_hbm, out_hbm,
               idx_v, rows_v, acc_sh, sem_g, sem_s, sem_i):
    cid = lax.axis_index("c")
    sid = lax.axis_index("s")
    wid = sid * 2 + cid

    @pl.when(sid < 8)
    def _():
        pltpu.sync_copy(zeros_hbm, acc_sh.at[pl.ds(sid * HALF, HALF)])

    def idx_start(i, s):
        pltpu.async_copy(idx_hbm.at[wid, i], idx_v.at[s], sem_i.at[s])

    def idx_wait(i, s):
        pltpu.make_async_copy(idx_hbm.at[wid, i], idx_v.at[s],
                              sem_i.at[s]).wait()

    def gather(i, b):
        pltpu.async_copy(g_hbm.at[idx_v.at[lax.rem(i, NIDX), 0]],
                         rows_v.at[b], sem_g.at[b])

    def wait_g(i, b):
        pltpu.make_async_copy(g_hbm.at[idx_v.at[lax.rem(i, NIDX), 0]],
                              rows_v.at[b], sem_g.at[b]).wait()

    def scat(i, b):
        pltpu.async_copy(rows_v.at[b],
                         acc_sh.at[idx_v.at[lax.rem(i, NIDX), 1]],
                         sem_s.at[b], add=True)

    def wait_s(i, b):
        pltpu.make_async_copy(rows_v.at[b],
                              acc_sh.at[idx_v.at[lax.rem(i, NIDX), 1]],
                              sem_s.at[b]).wait()

    for j in range(4):
        idx_start(j, j)
    for j in range(2):
        idx_wait(j, j)
        gather(j, j)
    plsc.subcore_barrier()

    def body(i, carry):
        b = lax.rem(i, NBUF)
        wait_g(i, b)
        scat(i, b)

        @pl.when(i >= 1)
        def _():
            wait_s(i - 1, lax.rem(i - 1, NBUF))

        @pl.when(i + 4 < C)
        def _():
            idx_start(i + 4, lax.rem(i + 4, NIDX))

        @pl.when(i + 2 < C)
        def _():
            idx_wait(i + 2, lax.rem(i + 2, NIDX))
            gather(i + 2, lax.rem(i + 2, NBUF))

        return carry

    lax.fori_loop(0, C, body, 0)
    wait_s(C - 1, lax.rem(C - 1, NBUF))
    plsc.subcore_barrier()

    @pl.when(sid < 8)
    def _():
        pltpu.sync_copy(acc_sh.at[pl.ds(sid * HALF, HALF)],
                        out_hbm.at[cid, pl.ds(sid * HALF, HALF)])


_scat_call = functools.partial(
    pl.kernel,
    out_type=jax.ShapeDtypeStruct((2, N_PAD, D), jnp.float32),
    mesh=_mesh,
    scratch_types=[
        pltpu.VMEM((NIDX, 2, K), jnp.int32),
        pltpu.VMEM((NBUF, K, D), jnp.float32),
        pltpu.VMEM_SHARED((N_ACC, D), jnp.float32),
        pltpu.SemaphoreType.DMA((NBUF,)),
        pltpu.SemaphoreType.DMA((NBUF,)),
        pltpu.SemaphoreType.DMA((NIDX,)),
    ],
)(_scat_body)


RB = 512
_GRID = N_PAD // RB


def _mm1_body(deg_ref, x_ref, w_ref, g_ref, dinv_ref):
    deg = deg_ref[0] + deg_ref[1] + 1.0
    dinv = lax.rsqrt(deg)
    h = jnp.dot(x_ref[...], w_ref[...], preferred_element_type=jnp.float32)
    g_ref[...] = h * dinv
    dinv_ref[...] = dinv


def _mm1(deg_p, x_p, W1):
    return pl.pallas_call(
        _mm1_body,
        grid=(_GRID,),
        in_specs=[
            pl.BlockSpec((2, RB, 1), lambda i: (0, i, 0)),
            pl.BlockSpec((RB, D), lambda i: (i, 0)),
            pl.BlockSpec((D, D), lambda i: (0, 0)),
        ],
        out_specs=[
            pl.BlockSpec((RB, D), lambda i: (i, 0)),
            pl.BlockSpec((RB, 1), lambda i: (i, 0)),
        ],
        out_shape=[
            jax.ShapeDtypeStruct((N_PAD, D), jnp.float32),
            jax.ShapeDtypeStruct((N_PAD, 1), jnp.float32),
        ],
    )(deg_p, x_p, W1)


def _mid_body(p_ref, g_ref, dinv_ref, b_ref, a_ref, w_ref, g2_ref):
    dinv = dinv_ref[...]
    s = dinv * (p_ref[0] + p_ref[1] + g_ref[...]) + b_ref[...]
    aa = a_ref[0, 0]
    z = jnp.where(s >= 0, s, aa * s)
    h2 = jnp.dot(z, w_ref[...], preferred_element_type=jnp.float32)
    g2_ref[...] = h2 * dinv


def _mid(p1, g1, dinv, b1r, ar, W2):
    return pl.pallas_call(
        _mid_body,
        grid=(_GRID,),
        in_specs=[
            pl.BlockSpec((2, RB, D), lambda i: (0, i, 0)),
            pl.BlockSpec((RB, D), lambda i: (i, 0)),
            pl.BlockSpec((RB, 1), lambda i: (i, 0)),
            pl.BlockSpec((1, D), lambda i: (0, 0)),
            pl.BlockSpec((1, 1), lambda i: (0, 0)),
            pl.BlockSpec((D, D), lambda i: (0, 0)),
        ],
        out_specs=pl.BlockSpec((RB, D), lambda i: (i, 0)),
        out_shape=jax.ShapeDtypeStruct((N_PAD, D), jnp.float32),
    )(p1, g1, dinv, b1r, ar, W2)


def _fin_body(p_ref, g_ref, dinv_ref, b_ref, a_ref, out_ref):
    dinv = dinv_ref[...]
    s = dinv * (p_ref[0] + p_ref[1] + g_ref[...]) + b_ref[...]
    aa = a_ref[0, 0]
    out_ref[...] = jnp.where(s >= 0, s, aa * s)


def _fin(p2, g2, dinv, b2r, ar):
    return pl.pallas_call(
        _fin_body,
        grid=(_GRID,),
        in_specs=[
            pl.BlockSpec((2, RB, D), lambda i: (0, i, 0)),
            pl.BlockSpec((RB, D), lambda i: (i, 0)),
            pl.BlockSpec((RB, 1), lambda i: (i, 0)),
            pl.BlockSpec((1, D), lambda i: (0, 0)),
            pl.BlockSpec((1, 1), lambda i: (0, 0)),
        ],
        out_specs=pl.BlockSpec((RB, D), lambda i: (i, 0)),
        out_shape=jax.ShapeDtypeStruct((N_PAD, D), jnp.float32),
    )(p2, g2, dinv, b2r, ar)


def kernel(x, edge_index, W1, b1, W2, b2, a):
    src = edge_index[0]
    dst = edge_index[1]
    pad_idx = jnp.full((E_PAD - E,), N, dtype=jnp.int32)
    src_p = jnp.concatenate([src, pad_idx]).reshape(NW, C, K)
    dst_p = jnp.concatenate([dst, pad_idx]).reshape(NW, C, K)
    idx_p = jnp.stack([src_p, dst_p], axis=2)
    x_p = jnp.concatenate([x, jnp.zeros((N_PAD - N, D), x.dtype)])
    zerosN = jnp.zeros((N_PAD,), jnp.float32)
    zerosD = jnp.zeros((HALF, D), jnp.float32)
    b1r = b1.reshape(1, D)
    b2r = b2.reshape(1, D)
    ar = a.reshape(1, 1)

    deg_p = _deg_call(dst_p, zerosN).reshape(2, N_PAD, 1)
    g1, dinv = _mm1(deg_p, x_p, W1)
    p1 = _scat_call(g1, idx_p, zerosD)
    g2 = _mid(p1, g1, dinv, b1r, ar, W2)
    p2 = _scat_call(g2, idx_p, zerosD)
    out = _fin(p2, g2, dinv, b2r, ar)
    return out[:N]
